# baseline (speedup 1.0000x reference)
import functools

import jax
import jax.numpy as jnp
from jax import lax
from jax.experimental import pallas as pl
from jax.experimental.pallas import tpu as pltpu
from jax.experimental.pallas import tpu_sc as plsc

D = 128
S = 8
B = 16
N = 64
K = 200
LP = 50
NB = 1024
NP = 32

NCH = 8
NW = 32
R = B * N * K
C = 400

T = 1 + NP + N * K + N * S + N * LP
H0 = 1 + NP
S0 = H0 + N * K
Q0 = S0 + N * S

_C0 = 0.9999999788684018
_C1 = -0.499999241515366
_C2 = 0.04166389745641559
_C3 = -0.001385552366074646
_C4 = 2.318830153672049e-05


def _cospoly(y):
    q = y * y
    return _C0 + q * (_C1 + q * (_C2 + q * (_C3 + q * _C4)))


def _prep_body(hv_ref, z_ref, idx_ref):
    v = hv_ref[0, 0]
    mu = jnp.sum(v, axis=0, keepdims=True) / K
    d = v - mu
    var = jnp.sum(d * d, axis=0, keepdims=True) / (K - 1)
    sig = jnp.sqrt(var)
    z = jnp.clip(d / (sig + 1e-6), -5.0, 5.0)
    idx = jnp.clip(jnp.floor((z + 5.0) / 10.0 * (NB - 1)), 0, NB - 1)
    z_ref[0, 0] = z
    idx_ref[0, 0] = idx.astype(jnp.int32)


def _mega_body(z4_ref, ht4_ref, qt4_ref, ed_ref, ve_ref, snr_ref, pos_ref,
               head_ref, w1_ref, w2_ref, qb_ref, sw_ref, vw_ref, sbvb_ref,
               valw_ref, valb_ref, f_ref, sh_ref, out_ref, te_ref):
    dn = (((1,), (1,)), ((), ()))
    f128 = f_ref[...]
    shift = sh_ref[...]
    valw = valw_ref[...]
    valb = valb_ref[...]

    out_ref[0, 0:H0, :] = head_ref[...]

    for c in range(N // NCH):
        for j in range(NCH):
            n = c * NCH + j
            z = z4_ref[0, c, :, j:j + 1]
            t = ht4_ref[0, c, :, j:j + 1]
            te = _cospoly(t * f128 + shift)
            ed = ed_ref[0, n * K:(n + 1) * K, :]
            out_ref[0, H0 + n * K:H0 + (n + 1) * K, :] = \
                z * valw + valb + te + ed

    ve = ve_ref[0]

    vt = lax.dot_general(ve, vw_ref[...], dn,
                         preferred_element_type=jnp.float32)
    r8 = (lax.broadcasted_iota(jnp.int32, (N * S, N), 0) // S
          == lax.broadcasted_iota(jnp.int32, (N * S, N), 1)
          ).astype(jnp.float32)
    sc = (lax.dot_general(snr_ref[0], sw_ref[...], dn,
                          preferred_element_type=jnp.float32)
          + jnp.dot(r8, vt, preferred_element_type=jnp.float32)
          + pos_ref[...] + sbvb_ref[...])
    out_ref[0, S0:S0 + N * S, :] = sc

    for c in range(N // NCH):
        for j in range(NCH):
            n = c * NCH + j
            t = qt4_ref[0, c, :, j:j + 1]
            te_ref[n * LP:(n + 1) * LP, :] = _cospoly(t * f128 + shift)
    a = lax.dot_general(ve, w1_ref[...], dn,
                        preferred_element_type=jnp.float32) + qb_ref[...]
    r50 = (lax.broadcasted_iota(jnp.int32, (N * LP, N), 0) // LP
           == lax.broadcasted_iota(jnp.int32, (N * LP, N), 1)
           ).astype(jnp.float32)
    q = (lax.dot_general(te_ref[...], w2_ref[...], dn,
                         preferred_element_type=jnp.float32)
         + jnp.dot(r50, a, preferred_element_type=jnp.float32))
    out_ref[0, Q0:Q0 + N * LP, :] = q


def _make_sc_gather():
    mesh = plsc.VectorSubcoreMesh(core_axis_name="c", subcore_axis_name="s")
    rpw = R // NW
    nchunk = rpw // C

    @functools.partial(
        pl.kernel, mesh=mesh,
        out_type=jax.ShapeDtypeStruct((R, D), jnp.float32),
        scratch_types=[
            pltpu.VMEM((C,), jnp.int32),
            pltpu.VMEM((C,), jnp.int32),
            pltpu.VMEM((C, D), jnp.float32),
            pltpu.VMEM((C, D), jnp.float32),
            pltpu.SemaphoreType.DMA,
            pltpu.SemaphoreType.DMA,
            pltpu.SemaphoreType.DMA,
            pltpu.SemaphoreType.DMA,
            pltpu.SemaphoreType.DMA,
            pltpu.SemaphoreType.DMA,
        ],
    )
    def _sc_gather(idx_hbm, table_hbm, out_hbm,
                   idx_v0, idx_v1, rows_v0, rows_v1,
                   isem0, isem1, gsem0, gsem1, osem0, osem1):
        wid = lax.axis_index("s") * 2 + lax.axis_index("c")
        slots = ((idx_v0, rows_v0, isem0, gsem0, osem0),
                 (idx_v1, rows_v1, isem1, gsem1, osem1))

        def start_load(i, s):
            idx_v, _, isem, _, _ = slots[s]
            pltpu.async_copy(idx_hbm.at[pl.ds(wid * rpw + i * C, C)],
                             idx_v, isem)

        def gather(i, s):
            idx_v, rows_v, isem, gsem, _ = slots[s]
            pltpu.make_async_copy(idx_hbm.at[pl.ds(wid * rpw + i * C, C)],
                                  idx_v, isem).wait()
            pltpu.async_copy(table_hbm.at[idx_v], rows_v, gsem)

        def store(i, s):
            idx_v, rows_v, _, gsem, osem = slots[s]
            pltpu.make_async_copy(table_hbm.at[idx_v], rows_v, gsem).wait()
            pltpu.async_copy(rows_v, out_hbm.at[pl.ds(wid * rpw + i * C, C)],
                             osem)

        def wait_store(i, s):
            _, rows_v, _, _, osem = slots[s]
            pltpu.make_async_copy(rows_v, out_hbm.at[pl.ds(
                wid * rpw + i * C, C)], osem).wait()

        start_load(0, 0)
        start_load(1, 1)

        def body(g, carry):
            e = g * 2

            gather(e, 0)
            store(e, 0)
            gather(e + 1, 1)
            wait_store(e, 0)

            @pl.when(e + 2 < nchunk)
            def _():
                start_load(e + 2, 0)

            store(e + 1, 1)
            wait_store(e + 1, 1)

            @pl.when(e + 3 < nchunk)
            def _():
                start_load(e + 3, 1)
            return carry

        lax.fori_loop(0, nchunk // 2, body, 0)

    return _sc_gather


def kernel(scale_nodes, var_emb, query_times, hist_vals, hist_times,
           scale_pos, var_W, var_b, scale_W, scale_b, query_W, query_b,
           cls_token, prompts, bucket_table, val_W, val_b):
    half = D // 2
    freqs = jnp.exp(-jnp.log(10000.0)
                    * jnp.arange(half, dtype=jnp.float32) / half)
    f128 = jnp.concatenate([freqs, freqs])[None, :]
    shift = jnp.concatenate([jnp.full((half,), -jnp.pi / 2, jnp.float32),
                             jnp.zeros((half,), jnp.float32)]
                            )[None, :]

    hv4 = hist_vals[..., 0].reshape(B, N // NCH, NCH, K).transpose(0, 1, 3, 2)
    ht4 = hist_times.reshape(B, N // NCH, NCH, K).transpose(0, 1, 3, 2)

    z4, idx4 = pl.pallas_call(
        _prep_body,
        grid=(B, N // NCH),
        in_specs=[pl.BlockSpec((1, 1, K, NCH), lambda b, c: (b, c, 0, 0))],
        out_specs=[
            pl.BlockSpec((1, 1, K, NCH), lambda b, c: (b, c, 0, 0)),
            pl.BlockSpec((1, 1, K, NCH), lambda b, c: (b, c, 0, 0)),
        ],
        out_shape=[
            jax.ShapeDtypeStruct((B, N // NCH, K, NCH), jnp.float32),
            jax.ShapeDtypeStruct((B, N // NCH, K, NCH), jnp.int32),
        ],
    )(hv4)

    idx_flat = idx4.transpose(0, 1, 3, 2).reshape(-1)

    emb_disc = _make_sc_gather()(idx_flat, bucket_table)

    qt4 = query_times.reshape(B, N // NCH, NCH, LP).transpose(0, 1, 3, 2)
    snr = scale_nodes.transpose(1, 2, 0, 3).reshape(B, N * S, D)
    pos_tile = jnp.tile(scale_pos, (N, 1))
    head = jnp.concatenate([cls_token[0], prompts], axis=0)

    tokens = pl.pallas_call(
        _mega_body,
        grid=(B,),
        in_specs=[
            pl.BlockSpec((1, N // NCH, K, NCH), lambda b: (b, 0, 0, 0)),
            pl.BlockSpec((1, N // NCH, K, NCH), lambda b: (b, 0, 0, 0)),
            pl.BlockSpec((1, N // NCH, LP, NCH), lambda b: (b, 0, 0, 0)),
            pl.BlockSpec((1, N * K, D), lambda b: (b, 0, 0)),
            pl.BlockSpec((1, N, D), lambda b: (b, 0, 0)),
            pl.BlockSpec((1, N * S, D), lambda b: (b, 0, 0)),
            pl.BlockSpec((N * S, D), lambda b: (0, 0)),
            pl.BlockSpec((H0, D), lambda b: (0, 0)),
            pl.BlockSpec((D, D), lambda b: (0, 0)),
            pl.BlockSpec((D, D), lambda b: (0, 0)),
            pl.BlockSpec((1, D), lambda b: (0, 0)),
            pl.BlockSpec((D, D), lambda b: (0, 0)),
            pl.BlockSpec((D, D), lambda b: (0, 0)),
            pl.BlockSpec((1, D), lambda b: (0, 0)),
            pl.BlockSpec((1, D), lambda b: (0, 0)),
            pl.BlockSpec((1, D), lambda b: (0, 0)),
            pl.BlockSpec((1, D), lambda b: (0, 0)),
            pl.BlockSpec((1, D), lambda b: (0, 0)),
        ],
        out_specs=pl.BlockSpec((1, T, D), lambda b: (b, 0, 0)),
        out_shape=jax.ShapeDtypeStruct((B, T, D), jnp.float32),
        scratch_shapes=[pltpu.VMEM((N * LP, D), jnp.float32)],
    )(z4, ht4, qt4, emb_disc.reshape(B, N * K, D), var_emb, snr, pos_tile,
      head, query_W[:, :D], query_W[:, D:], query_b[None, :], scale_W,
      var_W, (scale_b + var_b)[None, :], val_W[:, 0][None, :],
      val_b[None, :], f128, shift)

    attn_mask = jnp.ones((B, T), dtype=jnp.int32)
    return tokens, attn_mask
